# Initial kernel scaffold; baseline (speedup 1.0000x reference)
import functools

import jax
import jax.numpy as jnp
from jax import lax
from jax.experimental import pallas as pl
from jax.experimental.pallas import tpu as pltpu
from jax.experimental.pallas import tpu_sc as plsc

N = 10000
D = 128
E = 320000
NC = 2
NS = 16
NW = NC * NS
EPW = E // NW
BLK = 200
NBLK = EPW // BLK
ROWS_PER_TILE = N // NS
LANES = 16


def _sc_aggregate(vecs, src, dst, ew):
    mesh = plsc.VectorSubcoreMesh(
        core_axis_name="c", subcore_axis_name="s",
        num_cores=NC, num_subcores=NS)

    @functools.partial(
        pl.kernel,
        out_type=jax.ShapeDtypeStruct((NC, N, D), jnp.float32),
        mesh=mesh,
        scratch_types=[
            pltpu.VMEM((BLK,), jnp.int32),
            pltpu.VMEM((BLK,), jnp.int32),
            pltpu.VMEM((BLK,), jnp.float32),
            pltpu.VMEM((BLK, D), jnp.float32),
            pltpu.VMEM_SHARED((N, D), jnp.float32),
            pltpu.SemaphoreType.DMA,
        ],
    )
    def agg_kernel(vecs_hbm, src_hbm, dst_hbm, ew_hbm, out_hbm,
                   src_v, dst_v, w_v, rows_v, acc_sh, sem):
        cid = lax.axis_index("c")
        sid = lax.axis_index("s")
        wid = cid * NS + sid

        zero = jnp.zeros((LANES,), jnp.float32)

        def zbody(i, _):
            for c in range(D // LANES):
                rows_v[i, pl.ds(c * LANES, LANES)] = zero
            return 0

        lax.fori_loop(0, BLK, zbody, 0)
        my_row0 = sid * ROWS_PER_TILE
        pltpu.sync_copy(rows_v, acc_sh.at[pl.ds(my_row0, BLK)])
        pltpu.sync_copy(rows_v, acc_sh.at[pl.ds(my_row0 + BLK, BLK)])
        pltpu.sync_copy(rows_v, acc_sh.at[pl.ds(my_row0 + 2 * BLK, BLK)])
        pltpu.sync_copy(rows_v.at[pl.ds(0, ROWS_PER_TILE - 3 * BLK)],
                        acc_sh.at[pl.ds(my_row0 + 3 * BLK,
                                        ROWS_PER_TILE - 3 * BLK)])
        plsc.subcore_barrier()

        def block(b, _):
            base = wid * EPW + b * BLK
            pltpu.sync_copy(src_hbm.at[pl.ds(base, BLK)], src_v)
            pltpu.sync_copy(dst_hbm.at[pl.ds(base, BLK)], dst_v)
            pltpu.sync_copy(ew_hbm.at[pl.ds(base, BLK)], w_v)
            pltpu.async_copy(vecs_hbm.at[src_v], rows_v, sem).wait()

            def escale(e, _):
                wsplat = plsc.load_gather(
                    w_v, [jnp.broadcast_to(e, (LANES,))])
                for c in range(D // LANES):
                    sl = pl.ds(c * LANES, LANES)
                    rows_v[e, sl] = rows_v[e, sl] * wsplat
                return 0

            lax.fori_loop(0, BLK, escale, 0)
            pltpu.sync_copy(rows_v, acc_sh.at[dst_v], add=True)
            return 0

        lax.fori_loop(0, NBLK, block, 0)
        plsc.subcore_barrier()
        pltpu.sync_copy(acc_sh.at[pl.ds(my_row0, ROWS_PER_TILE)],
                        out_hbm.at[cid, pl.ds(my_row0, ROWS_PER_TILE)])

    return agg_kernel(vecs, src, dst, ew)


def _tc_dense(vecs, p0, p1, W0, b0, off0, sc0, W1, b1, off1, sc1):
    BN = 1000
    grid = (N // BN,)

    def body(x_ref, a0_ref, a1_ref, W0_ref, b0_ref, off0_ref, sc0_ref,
             W1_ref, b1_ref, off1_ref, sc1_ref, o_ref):
        def f(v, W, b, off, sc):
            vw = jnp.dot(v, W, preferred_element_type=jnp.float32) + b
            vw = jnp.maximum(vw, 0.0)
            mean = jnp.mean(vw, axis=1, keepdims=True)
            var = jnp.mean((vw - mean) ** 2, axis=1, keepdims=True)
            return (vw - mean) * lax.rsqrt(var + 1e-9) * sc + off

        x = x_ref[...]
        a = a0_ref[...] + a1_ref[...]
        o_ref[...] = (f(x, W0_ref[...], b0_ref[...], off0_ref[...],
                        sc0_ref[...])
                      + f(a, W1_ref[...], b1_ref[...], off1_ref[...],
                          sc1_ref[...]))

    row_spec = pl.BlockSpec((BN, D), lambda i: (i, 0))
    full = lambda shape: pl.BlockSpec(shape, lambda i: (0,) * len(shape))
    return pl.pallas_call(
        body,
        grid=grid,
        in_specs=[row_spec, row_spec, row_spec,
                  full((D, D)), full((1, D)), full((1, D)), full((1, D)),
                  full((D, D)), full((1, D)), full((1, D)), full((1, D))],
        out_specs=row_spec,
        out_shape=jax.ShapeDtypeStruct((N, D), jnp.float32),
    )(vecs, p0, p1, W0, b0, off0, sc0, W1, b1, off1, sc1)


def kernel(vecs, edge_index, edge_weight, W0, b0, off0, sc0,
           W1, b1, off1, sc1):
    src = edge_index[0]
    dst = edge_index[1]
    parts = _sc_aggregate(vecs, src, dst, edge_weight)
    return _tc_dense(vecs, parts[0], parts[1],
                     W0, b0.reshape(1, D), off0, sc0,
                     W1, b1.reshape(1, D), off1, sc1)


# speedup vs baseline: 4.4740x; 4.4740x over previous
import functools

import jax
import jax.numpy as jnp
from jax import lax
from jax.experimental import pallas as pl
from jax.experimental.pallas import tpu as pltpu
from jax.experimental.pallas import tpu_sc as plsc

N = 10000
D = 128
E = 320000
NC = 2
NS = 16
NW = NC * NS
EPW = E // NW
BLK = 80
NBLK = EPW // BLK
ROWS_PER_TILE = 624
TAIL_ROWS = N - NS * ROWS_PER_TILE
LANES = 16


def _sc_aggregate(vecs, src, dst, ew):
    mesh = plsc.VectorSubcoreMesh(
        core_axis_name="c", subcore_axis_name="s",
        num_cores=NC, num_subcores=NS)

    @functools.partial(
        pl.kernel,
        out_type=jax.ShapeDtypeStruct((NC, N, D), jnp.float32),
        mesh=mesh,
        scratch_types=[
            pltpu.VMEM((BLK,), jnp.int32),
            pltpu.VMEM((BLK,), jnp.int32),
            pltpu.VMEM((BLK,), jnp.float32),
            pltpu.VMEM((BLK, D), jnp.float32),
            pltpu.VMEM_SHARED((N, D), jnp.float32),
            pltpu.SemaphoreType.DMA,
        ],
    )
    def agg_kernel(vecs_hbm, src_hbm, dst_hbm, ew_hbm, out_hbm,
                   src_v, dst_v, w_v, rows_v, acc_sh, sem):
        cid = lax.axis_index("c")
        sid = lax.axis_index("s")
        wid = cid * NS + sid

        zero = jnp.zeros((LANES,), jnp.float32)

        def zbody(i, _):
            for c in range(D // LANES):
                rows_v[i, pl.ds(c * LANES, LANES)] = zero
            return 0

        lax.fori_loop(0, BLK, zbody, 0)
        my_row0 = sid * ROWS_PER_TILE

        def zcopy(i, _):
            pltpu.sync_copy(rows_v, acc_sh.at[pl.ds(my_row0 + i * BLK, BLK)])
            return 0

        lax.fori_loop(0, ROWS_PER_TILE // BLK, zcopy, 0)
        rem = ROWS_PER_TILE % BLK
        pltpu.sync_copy(
            rows_v.at[pl.ds(0, rem)],
            acc_sh.at[pl.ds(my_row0 + (ROWS_PER_TILE // BLK) * BLK, rem)])

        @pl.when(sid == NS - 1)
        def _zero_tail():
            pltpu.sync_copy(rows_v.at[pl.ds(0, TAIL_ROWS)],
                            acc_sh.at[pl.ds(NS * ROWS_PER_TILE, TAIL_ROWS)])

        plsc.subcore_barrier()

        def block(b, _):
            base = wid * EPW + b * BLK
            pltpu.sync_copy(src_hbm.at[pl.ds(base, BLK)], src_v)
            pltpu.sync_copy(dst_hbm.at[pl.ds(base, BLK)], dst_v)
            pltpu.sync_copy(ew_hbm.at[pl.ds(base, BLK)], w_v)
            pltpu.async_copy(vecs_hbm.at[src_v], rows_v, sem).wait()

            dnums = lax.GatherDimensionNumbers(
                offset_dims=(), collapsed_slice_dims=(0,),
                start_index_map=(0,))

            def escale(g, _):
                wgrp = w_v[pl.ds(g * LANES, LANES)]
                for j in range(LANES):
                    idx = jnp.full((LANES, 1), j, jnp.int32)
                    wsplat = lax.gather(
                        wgrp, idx, dnums, (1,),
                        mode=lax.GatherScatterMode.PROMISE_IN_BOUNDS)
                    e = g * LANES + j
                    for c in range(D // LANES):
                        sl = pl.ds(c * LANES, LANES)
                        rows_v[e, sl] = rows_v[e, sl] * wsplat
                return 0

            lax.fori_loop(0, BLK // LANES, escale, 0)
            pltpu.sync_copy(rows_v, acc_sh.at[dst_v], add=True)
            return 0

        lax.fori_loop(0, NBLK, block, 0)
        plsc.subcore_barrier()
        pltpu.sync_copy(acc_sh.at[pl.ds(my_row0, ROWS_PER_TILE)],
                        out_hbm.at[cid, pl.ds(my_row0, ROWS_PER_TILE)])

        @pl.when(sid == NS - 1)
        def _copy_tail():
            pltpu.sync_copy(
                acc_sh.at[pl.ds(NS * ROWS_PER_TILE, TAIL_ROWS)],
                out_hbm.at[cid, pl.ds(NS * ROWS_PER_TILE, TAIL_ROWS)])

    return agg_kernel(vecs, src, dst, ew)


def _tc_dense(vecs, p0, p1, W0, b0, off0, sc0, W1, b1, off1, sc1):
    BN = 1000
    grid = (N // BN,)

    def body(x_ref, a0_ref, a1_ref, W0_ref, b0_ref, off0_ref, sc0_ref,
             W1_ref, b1_ref, off1_ref, sc1_ref, o_ref):
        def f(v, W, b, off, sc):
            vw = jnp.dot(v, W, preferred_element_type=jnp.float32) + b
            vw = jnp.maximum(vw, 0.0)
            mean = jnp.mean(vw, axis=1, keepdims=True)
            var = jnp.mean((vw - mean) ** 2, axis=1, keepdims=True)
            return (vw - mean) * lax.rsqrt(var + 1e-9) * sc + off

        x = x_ref[...]
        a = a0_ref[...] + a1_ref[...]
        o_ref[...] = (f(x, W0_ref[...], b0_ref[...], off0_ref[...],
                        sc0_ref[...])
                      + f(a, W1_ref[...], b1_ref[...], off1_ref[...],
                          sc1_ref[...]))

    row_spec = pl.BlockSpec((BN, D), lambda i: (i, 0))
    full = lambda shape: pl.BlockSpec(shape, lambda i: (0,) * len(shape))
    return pl.pallas_call(
        body,
        grid=grid,
        in_specs=[row_spec, row_spec, row_spec,
                  full((D, D)), full((1, D)), full((1, D)), full((1, D)),
                  full((D, D)), full((1, D)), full((1, D)), full((1, D))],
        out_specs=row_spec,
        out_shape=jax.ShapeDtypeStruct((N, D), jnp.float32),
    )(vecs, p0, p1, W0, b0, off0, sc0, W1, b1, off1, sc1)


def kernel(vecs, edge_index, edge_weight, W0, b0, off0, sc0,
           W1, b1, off1, sc1):
    src = edge_index[0]
    dst = edge_index[1]
    parts = _sc_aggregate(vecs, src, dst, edge_weight)
    return _tc_dense(vecs, parts[0], parts[1],
                     W0, b0.reshape(1, D), off0, sc0,
                     W1, b1.reshape(1, D), off1, sc1)
